# Initial kernel scaffold; baseline (speedup 1.0000x reference)
import jax
import jax.numpy as jnp
from jax.experimental import pallas as pl


def kernel(time, data, org_edge_index, emb, W_lin, att_i, att_j, att_em_i, att_em_j, gl_bias, bn1_g, bn1_b, bn2_g, bn2_b, out_W, out_b, tr_layers, fc_W, fc_b):
    raise NotImplementedError("write your pallas kernel here")


# speedup vs baseline: 286.5162x; 286.5162x over previous
import functools

import jax
import jax.numpy as jnp
from jax.experimental import pallas as pl

NODE_NUM = 512
DIM = 64
INPUT_DIM = 10
TOPK = 20
BATCH = 64
HEADS = 2
NHEAD = 4
NLAYERS = 6
DFF = 2048
CLASSES = 5
EPS = 1e-5
BN_COUNT = float(BATCH * NODE_NUM)


def _nt(a, b):
    return jax.lax.dot_general(a, b, (((1,), (1,)), ((), ())),
                               preferred_element_type=jnp.float32)


def _nn(a, b):
    return jnp.dot(a, b, preferred_element_type=jnp.float32)


def _mask_kernel(cos_ref, m_ref):
    c = cos_ref[...]
    ji = jax.lax.broadcasted_iota(jnp.int32, (NODE_NUM, NODE_NUM), 1)
    ri = jax.lax.broadcasted_iota(jnp.int32, (NODE_NUM, NODE_NUM), 0)
    sel = jnp.zeros((NODE_NUM, NODE_NUM), jnp.float32)
    work = c
    big = jnp.int32(1 << 30)
    for _ in range(TOPK):
        mrow = jnp.max(work, axis=1, keepdims=True)
        eq = work == mrow
        idx = jnp.min(jnp.where(eq, ji, big), axis=1, keepdims=True)
        first = ji == idx
        sel = jnp.where(first, 1.0, sel)
        work = jnp.where(first, -jnp.inf, work)
    m_ref[...] = jnp.where(ji == ri, 1.0, sel)


def _gat_kernel(data_ref, wlin_ref, ait_ref, ajt_ref, emi_ref, emj_ref,
                emb_ref, mask_ref, bias_ref, out_ref, s_ref, q_ref):
    x = data_ref[0]
    xl = _nt(x, wlin_ref[...])
    emb = emb_ref[...]
    ai = _nt(xl, ait_ref[...]) + _nt(emb, emi_ref[...])
    ajr = _nt(ajt_ref[...], xl) + _nt(emj_ref[...], emb)
    msk = mask_ref[...]
    acc = None
    for h in range(HEADS):
        logit = ai[:, h:h + 1] + ajr[h:h + 1, :]
        logit = jnp.where(logit >= 0, logit, 0.2 * logit)
        logit = jnp.where(msk > 0, logit, -jnp.inf)
        mrow = jnp.max(logit, axis=1, keepdims=True)
        ex = jnp.exp(logit - mrow)
        att = ex / jnp.sum(ex, axis=1, keepdims=True)
        oh = _nn(att, xl[:, h * DIM:(h + 1) * DIM])
        acc = oh if acc is None else acc + oh
    out = acc * (1.0 / HEADS) + bias_ref[...]
    out_ref[0] = out
    s_ref[0] = jnp.sum(out, axis=0, keepdims=True)
    q_ref[0] = jnp.sum(out * out, axis=0, keepdims=True)


def _bn1_kernel(x_ref, s_ref, q_ref, g_ref, b_ref, emb_ref,
                o_ref, s2_ref, q2_ref):
    stot = jnp.sum(s_ref[...], axis=0)
    qtot = jnp.sum(q_ref[...], axis=0)
    mean = stot / BN_COUNT
    var = qtot / BN_COUNT - mean * mean
    scale = g_ref[...] / jnp.sqrt(var + EPS)
    shift = b_ref[...] - mean * scale
    o2 = jnp.maximum(x_ref[0] * scale + shift, 0.0)
    o3 = o2 * emb_ref[...]
    o_ref[0] = o3
    s2_ref[0] = jnp.sum(o3, axis=0, keepdims=True)
    q2_ref[0] = jnp.sum(o3 * o3, axis=0, keepdims=True)


def _bn2_kernel(x_ref, s_ref, q_ref, g_ref, b_ref, w_ref, ob_ref, r_ref):
    stot = jnp.sum(s_ref[...], axis=0)
    qtot = jnp.sum(q_ref[...], axis=0)
    mean = stot / BN_COUNT
    var = qtot / BN_COUNT - mean * mean
    scale = g_ref[...] / jnp.sqrt(var + EPS)
    shift = b_ref[...] - mean * scale
    o3 = jnp.maximum(x_ref[0] * scale + shift, 0.0)
    r_ref[0] = _nt(o3, w_ref[...]) + ob_ref[...]


def _ln(x, g, b):
    m = jnp.mean(x, axis=1, keepdims=True)
    v = jnp.mean((x - m) * (x - m), axis=1, keepdims=True)
    return (x - m) / jnp.sqrt(v + EPS) * g + b


def _enc_kernel(x_ref, wq_ref, bq_ref, wk_ref, bk_ref, wv_ref, bv_ref,
                wo_ref, bo_ref, w1_ref, b1_ref, w2_ref, b2_ref,
                g1_ref, n1_ref, g2_ref, n2_ref, *refs, first, last):
    if last:
        fcw_ref, fcb_ref, xo_ref, lg_ref = refs
    else:
        (xo_ref,) = refs
    n_tok = BATCH * INPUT_DIM
    x = x_ref[...]
    if first:
        ri = jax.lax.broadcasted_iota(jnp.int32, (n_tok, NODE_NUM), 0)
        x = x + jax.lax.rem(ri, INPUT_DIM).astype(jnp.float32)
    q = _nt(x, wq_ref[...]) + bq_ref[...]
    k = _nt(x, wk_ref[...]) + bk_ref[...]
    v = _nt(x, wv_ref[...]) + bv_ref[...]
    rb = jax.lax.broadcasted_iota(jnp.int32, (n_tok, n_tok), 0) // INPUT_DIM
    cb = jax.lax.broadcasted_iota(jnp.int32, (n_tok, n_tok), 1) // INPUT_DIM
    same = rb == cb
    dh = NODE_NUM // NHEAD
    inv = 1.0 / (dh ** 0.5)
    outs = []
    for h in range(NHEAD):
        sl = slice(h * dh, (h + 1) * dh)
        sc = _nt(q[:, sl], k[:, sl]) * inv
        sc = jnp.where(same, sc, -jnp.inf)
        mrow = jnp.max(sc, axis=1, keepdims=True)
        ex = jnp.exp(sc - mrow)
        att = ex / jnp.sum(ex, axis=1, keepdims=True)
        outs.append(_nn(att, v[:, sl]))
    o = jnp.concatenate(outs, axis=1)
    x = _ln(x + _nt(o, wo_ref[...]) + bo_ref[...], g1_ref[...], n1_ref[...])
    f = jnp.maximum(_nt(x, w1_ref[...]) + b1_ref[...], 0.0)
    x = _ln(x + _nt(f, w2_ref[...]) + b2_ref[...], g2_ref[...], n2_ref[...])
    xo_ref[...] = x
    if last:
        pr = jax.lax.broadcasted_iota(jnp.int32, (BATCH, n_tok), 0)
        pc = jax.lax.broadcasted_iota(jnp.int32, (BATCH, n_tok), 1)
        pool = jnp.where(pc // INPUT_DIM == pr, 1.0 / INPUT_DIM, 0.0)
        pooled = _nn(pool, x)
        lg_ref[...] = _nt(pooled, fcw_ref[...]) + fcb_ref[...]


def _full(shape):
    nd = len(shape)
    return pl.BlockSpec(shape, lambda b, _n=nd: (0,) * _n)


def kernel(time, data, org_edge_index, emb, W_lin, att_i, att_j, att_em_i,
           att_em_j, gl_bias, bn1_g, bn1_b, bn2_g, bn2_b, out_W, out_b,
           tr_layers, fc_W, fc_b):
    f32 = jnp.float32
    w = jax.lax.stop_gradient(emb)
    nrm = jnp.linalg.norm(w, axis=-1, keepdims=True)
    cos = (w @ w.T) / (nrm @ nrm.T)

    mask = pl.pallas_call(
        _mask_kernel,
        out_shape=jax.ShapeDtypeStruct((NODE_NUM, NODE_NUM), f32),
    )(cos)

    z = jnp.zeros((HEADS, HEADS * DIM), f32)
    ait = z
    ajt = z
    for h in range(HEADS):
        ait = ait.at[h, h * DIM:(h + 1) * DIM].set(att_i[0, h])
        ajt = ajt.at[h, h * DIM:(h + 1) * DIM].set(att_j[0, h])
    emi = att_em_i[0]
    emj = att_em_j[0]

    grid = (BATCH,)
    b_nd = pl.BlockSpec((1, NODE_NUM, DIM), lambda b: (b, 0, 0))
    b_part = pl.BlockSpec((1, 1, DIM), lambda b: (b, 0, 0))
    out1, s1, q1 = pl.pallas_call(
        _gat_kernel,
        grid=grid,
        in_specs=[pl.BlockSpec((1, NODE_NUM, INPUT_DIM), lambda b: (b, 0, 0)),
                  _full(W_lin.shape), _full(ait.shape), _full(ajt.shape),
                  _full(emi.shape), _full(emj.shape), _full(emb.shape),
                  _full((NODE_NUM, NODE_NUM)), _full((1, DIM))],
        out_specs=[b_nd, b_part, b_part],
        out_shape=[jax.ShapeDtypeStruct((BATCH, NODE_NUM, DIM), f32),
                   jax.ShapeDtypeStruct((BATCH, 1, DIM), f32),
                   jax.ShapeDtypeStruct((BATCH, 1, DIM), f32)],
    )(data, W_lin, ait, ajt, emi, emj, emb, mask, gl_bias.reshape(1, DIM))

    o3p, s2, q2 = pl.pallas_call(
        _bn1_kernel,
        grid=grid,
        in_specs=[b_nd, _full((BATCH, 1, DIM)), _full((BATCH, 1, DIM)),
                  _full((1, DIM)), _full((1, DIM)), _full(emb.shape)],
        out_specs=[b_nd, b_part, b_part],
        out_shape=[jax.ShapeDtypeStruct((BATCH, NODE_NUM, DIM), f32),
                   jax.ShapeDtypeStruct((BATCH, 1, DIM), f32),
                   jax.ShapeDtypeStruct((BATCH, 1, DIM), f32)],
    )(out1, s1, q1, bn1_g.reshape(1, DIM), bn1_b.reshape(1, DIM), emb)

    recon = pl.pallas_call(
        _bn2_kernel,
        grid=grid,
        in_specs=[b_nd, _full((BATCH, 1, DIM)), _full((BATCH, 1, DIM)),
                  _full((1, DIM)), _full((1, DIM)), _full(out_W.shape),
                  _full((1, INPUT_DIM))],
        out_specs=[pl.BlockSpec((1, NODE_NUM, INPUT_DIM), lambda b: (b, 0, 0))],
        out_shape=[jax.ShapeDtypeStruct((BATCH, NODE_NUM, INPUT_DIM), f32)],
    )(o3p, s2, q2, bn2_g.reshape(1, DIM), bn2_b.reshape(1, DIM), out_W,
      out_b.reshape(1, INPUT_DIM))[0]

    n_tok = BATCH * INPUT_DIM
    x = jnp.transpose(recon, (0, 2, 1)).reshape(n_tok, NODE_NUM)
    logits = None
    for li, p in enumerate(tr_layers):
        first = li == 0
        last = li == NLAYERS - 1
        args = [x, p["Wq"], p["bq"].reshape(1, -1), p["Wk"],
                p["bk"].reshape(1, -1), p["Wv"], p["bv"].reshape(1, -1),
                p["Wo"], p["bo"].reshape(1, -1), p["W1"],
                p["b1"].reshape(1, -1), p["W2"], p["b2"].reshape(1, -1),
                p["ln1_g"].reshape(1, -1), p["ln1_b"].reshape(1, -1),
                p["ln2_g"].reshape(1, -1), p["ln2_b"].reshape(1, -1)]
        out_shape = [jax.ShapeDtypeStruct((n_tok, NODE_NUM), f32)]
        if last:
            args += [fc_W, fc_b.reshape(1, CLASSES)]
            out_shape.append(jax.ShapeDtypeStruct((BATCH, CLASSES), f32))
        res = pl.pallas_call(
            functools.partial(_enc_kernel, first=first, last=last),
            out_shape=out_shape,
        )(*args)
        if last:
            logits = res[1]
        else:
            x = res[0]

    return recon, logits


# speedup vs baseline: 335.0341x; 1.1693x over previous
import functools

import jax
import jax.numpy as jnp
from jax.experimental import pallas as pl

NODE_NUM = 512
DIM = 64
INPUT_DIM = 10
TOPK = 20
BATCH = 64
HEADS = 2
NHEAD = 4
NLAYERS = 6
DFF = 2048
CLASSES = 5
EPS = 1e-5
BN_COUNT = float(BATCH * NODE_NUM)


def _nt(a, b):
    return jax.lax.dot_general(a, b, (((1,), (1,)), ((), ())),
                               preferred_element_type=jnp.float32)


def _nn(a, b):
    return jnp.dot(a, b, preferred_element_type=jnp.float32)


def _mask_kernel(cos_ref, m_ref):
    c = cos_ref[...]
    ji = jax.lax.broadcasted_iota(jnp.int32, (NODE_NUM, NODE_NUM), 1)
    ri = jax.lax.broadcasted_iota(jnp.int32, (NODE_NUM, NODE_NUM), 0)
    sel = jnp.zeros((NODE_NUM, NODE_NUM), jnp.float32)
    work = c
    big = jnp.int32(1 << 30)
    for _ in range(TOPK):
        mrow = jnp.max(work, axis=1, keepdims=True)
        eq = work == mrow
        idx = jnp.min(jnp.where(eq, ji, big), axis=1, keepdims=True)
        first = ji == idx
        sel = jnp.where(first, 1.0, sel)
        work = jnp.where(first, -jnp.inf, work)
    keep = (ji == ri) | (sel > 0)
    m_ref[...] = jnp.where(keep, 0.0, -jnp.inf)


GAT_BB = 2


def _gat_kernel(data_ref, wlin_ref, ait_ref, ajt_ref, emi_ref, emj_ref,
                emb_ref, mask_ref, bias_ref, out_ref, s_ref, q_ref):
    emb = emb_ref[...]
    msk = mask_ref[...]
    for bb in range(GAT_BB):
        x = data_ref[bb]
        xl = _nt(x, wlin_ref[...])
        ai = _nt(xl, ait_ref[...]) + _nt(emb, emi_ref[...])
        ajr = _nt(ajt_ref[...], xl) + _nt(emj_ref[...], emb)
        acc = None
        for h in range(HEADS):
            logit = ai[:, h:h + 1] + ajr[h:h + 1, :]
            logit = jnp.maximum(logit, 0.2 * logit) + msk
            mrow = jnp.max(logit, axis=1, keepdims=True)
            ex = jnp.exp(logit - mrow)
            inv = 1.0 / jnp.sum(ex, axis=1, keepdims=True)
            oh = _nn(ex, xl[:, h * DIM:(h + 1) * DIM]) * inv
            acc = oh if acc is None else acc + oh
        out = acc * (1.0 / HEADS) + bias_ref[...]
        out_ref[bb] = out
        s_ref[bb] = jnp.sum(out, axis=0, keepdims=True)
        q_ref[bb] = jnp.sum(out * out, axis=0, keepdims=True)


BN_BB = 4


def _bn1_kernel(x_ref, s_ref, q_ref, g_ref, b_ref, emb_ref,
                o_ref, s2_ref, q2_ref):
    stot = jnp.sum(s_ref[...], axis=0)
    qtot = jnp.sum(q_ref[...], axis=0)
    mean = stot / BN_COUNT
    var = qtot / BN_COUNT - mean * mean
    scale = g_ref[...] / jnp.sqrt(var + EPS)
    shift = b_ref[...] - mean * scale
    emb = emb_ref[...]
    for bb in range(BN_BB):
        o2 = jnp.maximum(x_ref[bb] * scale + shift, 0.0)
        o3 = o2 * emb
        o_ref[bb] = o3
        s2_ref[bb] = jnp.sum(o3, axis=0, keepdims=True)
        q2_ref[bb] = jnp.sum(o3 * o3, axis=0, keepdims=True)


def _bn2_kernel(x_ref, s_ref, q_ref, g_ref, b_ref, w_ref, ob_ref, r_ref):
    stot = jnp.sum(s_ref[...], axis=0)
    qtot = jnp.sum(q_ref[...], axis=0)
    mean = stot / BN_COUNT
    var = qtot / BN_COUNT - mean * mean
    scale = g_ref[...] / jnp.sqrt(var + EPS)
    shift = b_ref[...] - mean * scale
    for bb in range(BN_BB):
        o3 = jnp.maximum(x_ref[bb] * scale + shift, 0.0)
        r_ref[bb] = _nt(o3, w_ref[...]) + ob_ref[...]


def _ln(x, g, b):
    m = jnp.mean(x, axis=1, keepdims=True)
    v = jnp.mean((x - m) * (x - m), axis=1, keepdims=True)
    return (x - m) / jnp.sqrt(v + EPS) * g + b


def _enc_kernel(x_ref, wq_ref, bq_ref, wk_ref, bk_ref, wv_ref, bv_ref,
                wo_ref, bo_ref, w1_ref, b1_ref, w2_ref, b2_ref,
                g1_ref, n1_ref, g2_ref, n2_ref, *refs, first, last):
    if last:
        fcw_ref, fcb_ref, xo_ref, lg_ref = refs
    else:
        (xo_ref,) = refs
    n_tok = BATCH * INPUT_DIM
    x = x_ref[...]
    if first:
        ri = jax.lax.broadcasted_iota(jnp.int32, (n_tok, NODE_NUM), 0)
        x = x + jax.lax.rem(ri, INPUT_DIM).astype(jnp.float32)
    q = _nt(x, wq_ref[...]) + bq_ref[...]
    k = _nt(x, wk_ref[...]) + bk_ref[...]
    v = _nt(x, wv_ref[...]) + bv_ref[...]
    rb = jax.lax.broadcasted_iota(jnp.int32, (n_tok, n_tok), 0) // INPUT_DIM
    cb = jax.lax.broadcasted_iota(jnp.int32, (n_tok, n_tok), 1) // INPUT_DIM
    amask = jnp.where(rb == cb, 0.0, -jnp.inf)
    dh = NODE_NUM // NHEAD
    scale = 1.0 / (dh ** 0.5)
    outs = []
    for h in range(NHEAD):
        sl = slice(h * dh, (h + 1) * dh)
        sc = _nt(q[:, sl], k[:, sl]) * scale + amask
        mrow = jnp.max(sc, axis=1, keepdims=True)
        ex = jnp.exp(sc - mrow)
        inv = 1.0 / jnp.sum(ex, axis=1, keepdims=True)
        outs.append(_nn(ex, v[:, sl]) * inv)
    o = jnp.concatenate(outs, axis=1)
    x = _ln(x + _nt(o, wo_ref[...]) + bo_ref[...], g1_ref[...], n1_ref[...])
    f = jnp.maximum(_nt(x, w1_ref[...]) + b1_ref[...], 0.0)
    x = _ln(x + _nt(f, w2_ref[...]) + b2_ref[...], g2_ref[...], n2_ref[...])
    xo_ref[...] = x
    if last:
        pr = jax.lax.broadcasted_iota(jnp.int32, (BATCH, n_tok), 0)
        pc = jax.lax.broadcasted_iota(jnp.int32, (BATCH, n_tok), 1)
        pool = jnp.where(pc // INPUT_DIM == pr, 1.0 / INPUT_DIM, 0.0)
        pooled = _nn(pool, x)
        lg_ref[...] = _nt(pooled, fcw_ref[...]) + fcb_ref[...]


def _full(shape):
    nd = len(shape)
    return pl.BlockSpec(shape, lambda b, _n=nd: (0,) * _n)


def kernel(time, data, org_edge_index, emb, W_lin, att_i, att_j, att_em_i,
           att_em_j, gl_bias, bn1_g, bn1_b, bn2_g, bn2_b, out_W, out_b,
           tr_layers, fc_W, fc_b):
    f32 = jnp.float32
    w = jax.lax.stop_gradient(emb)
    nrm = jnp.linalg.norm(w, axis=-1, keepdims=True)
    cos = (w @ w.T) / (nrm @ nrm.T)

    mask = pl.pallas_call(
        _mask_kernel,
        out_shape=jax.ShapeDtypeStruct((NODE_NUM, NODE_NUM), f32),
    )(cos)

    z = jnp.zeros((HEADS, HEADS * DIM), f32)
    ait = z
    ajt = z
    for h in range(HEADS):
        ait = ait.at[h, h * DIM:(h + 1) * DIM].set(att_i[0, h])
        ajt = ajt.at[h, h * DIM:(h + 1) * DIM].set(att_j[0, h])
    emi = att_em_i[0]
    emj = att_em_j[0]

    g_nd = pl.BlockSpec((GAT_BB, NODE_NUM, DIM), lambda b: (b, 0, 0))
    g_part = pl.BlockSpec((GAT_BB, 1, DIM), lambda b: (b, 0, 0))
    out1, s1, q1 = pl.pallas_call(
        _gat_kernel,
        grid=(BATCH // GAT_BB,),
        in_specs=[pl.BlockSpec((GAT_BB, NODE_NUM, INPUT_DIM),
                               lambda b: (b, 0, 0)),
                  _full(W_lin.shape), _full(ait.shape), _full(ajt.shape),
                  _full(emi.shape), _full(emj.shape), _full(emb.shape),
                  _full((NODE_NUM, NODE_NUM)), _full((1, DIM))],
        out_specs=[g_nd, g_part, g_part],
        out_shape=[jax.ShapeDtypeStruct((BATCH, NODE_NUM, DIM), f32),
                   jax.ShapeDtypeStruct((BATCH, 1, DIM), f32),
                   jax.ShapeDtypeStruct((BATCH, 1, DIM), f32)],
    )(data, W_lin, ait, ajt, emi, emj, emb, mask, gl_bias.reshape(1, DIM))

    b_nd = pl.BlockSpec((BN_BB, NODE_NUM, DIM), lambda b: (b, 0, 0))
    b_part = pl.BlockSpec((BN_BB, 1, DIM), lambda b: (b, 0, 0))
    o3p, s2, q2 = pl.pallas_call(
        _bn1_kernel,
        grid=(BATCH // BN_BB,),
        in_specs=[b_nd, _full((BATCH, 1, DIM)), _full((BATCH, 1, DIM)),
                  _full((1, DIM)), _full((1, DIM)), _full(emb.shape)],
        out_specs=[b_nd, b_part, b_part],
        out_shape=[jax.ShapeDtypeStruct((BATCH, NODE_NUM, DIM), f32),
                   jax.ShapeDtypeStruct((BATCH, 1, DIM), f32),
                   jax.ShapeDtypeStruct((BATCH, 1, DIM), f32)],
    )(out1, s1, q1, bn1_g.reshape(1, DIM), bn1_b.reshape(1, DIM), emb)

    recon = pl.pallas_call(
        _bn2_kernel,
        grid=(BATCH // BN_BB,),
        in_specs=[b_nd, _full((BATCH, 1, DIM)), _full((BATCH, 1, DIM)),
                  _full((1, DIM)), _full((1, DIM)), _full(out_W.shape),
                  _full((1, INPUT_DIM))],
        out_specs=[pl.BlockSpec((BN_BB, NODE_NUM, INPUT_DIM),
                                lambda b: (b, 0, 0))],
        out_shape=[jax.ShapeDtypeStruct((BATCH, NODE_NUM, INPUT_DIM), f32)],
    )(o3p, s2, q2, bn2_g.reshape(1, DIM), bn2_b.reshape(1, DIM), out_W,
      out_b.reshape(1, INPUT_DIM))[0]

    n_tok = BATCH * INPUT_DIM
    x = jnp.transpose(recon, (0, 2, 1)).reshape(n_tok, NODE_NUM)
    logits = None
    for li, p in enumerate(tr_layers):
        first = li == 0
        last = li == NLAYERS - 1
        args = [x, p["Wq"], p["bq"].reshape(1, -1), p["Wk"],
                p["bk"].reshape(1, -1), p["Wv"], p["bv"].reshape(1, -1),
                p["Wo"], p["bo"].reshape(1, -1), p["W1"],
                p["b1"].reshape(1, -1), p["W2"], p["b2"].reshape(1, -1),
                p["ln1_g"].reshape(1, -1), p["ln1_b"].reshape(1, -1),
                p["ln2_g"].reshape(1, -1), p["ln2_b"].reshape(1, -1)]
        out_shape = [jax.ShapeDtypeStruct((n_tok, NODE_NUM), f32)]
        if last:
            args += [fc_W, fc_b.reshape(1, CLASSES)]
            out_shape.append(jax.ShapeDtypeStruct((BATCH, CLASSES), f32))
        res = pl.pallas_call(
            functools.partial(_enc_kernel, first=first, last=last),
            out_shape=out_shape,
        )(*args)
        if last:
            logits = res[1]
        else:
            x = res[0]

    return recon, logits
